# Initial kernel scaffold; baseline (speedup 1.0000x reference)
import jax
import jax.numpy as jnp
from jax.experimental import pallas as pl


def kernel(output, batch):
    raise NotImplementedError("write your pallas kernel here")


# speedup vs baseline: 4.9793x; 4.9793x over previous
import jax
import jax.numpy as jnp
from jax import lax
from jax.experimental import pallas as pl
from jax.experimental.pallas import tpu as pltpu
from jax.experimental.pallas import tpu_sc as plsc
import functools

N = 320000
D = 128
NSEG = 10000

NC = 2
NS = 16
NW = NC * NS
ROWS_PER_TILE = N // NW
CHUNK = 200
NCHUNK = ROWS_PER_TILE // CHUNK
NSEG_PAD = 10240
SEG_PER_TILE = NSEG_PAD // NS


def _sc_segment_partials(rows, ids, zeros):
    mesh = plsc.VectorSubcoreMesh(core_axis_name="c", subcore_axis_name="s")

    @functools.partial(
        pl.kernel,
        out_type=jax.ShapeDtypeStruct((NC, NSEG_PAD, D), jnp.float32),
        mesh=mesh,
        scratch_types=[
            pltpu.VMEM((CHUNK, D), jnp.float32),
            pltpu.VMEM((CHUNK,), jnp.int32),
            pltpu.VMEM_SHARED((NSEG_PAD, D), jnp.float32),
        ],
    )
    def body(rows_hbm, ids_hbm, zeros_hbm, part_hbm, rows_v, idx_v, acc):
        cid = lax.axis_index("c")
        sid = lax.axis_index("s")
        base_row = (cid * NS + sid) * ROWS_PER_TILE
        zbase = sid * SEG_PER_TILE

        pltpu.sync_copy(zeros_hbm.at[pl.ds(zbase, SEG_PER_TILE)],
                        acc.at[pl.ds(zbase, SEG_PER_TILE)])
        plsc.subcore_barrier()

        def chunk_body(i, carry):
            off = base_row + i * CHUNK
            pltpu.sync_copy(ids_hbm.at[pl.ds(off, CHUNK)], idx_v)
            pltpu.sync_copy(rows_hbm.at[pl.ds(off, CHUNK)], rows_v)
            pltpu.sync_copy(rows_v, acc.at[idx_v], add=True)
            return carry

        lax.fori_loop(0, NCHUNK, chunk_body, 0)
        plsc.subcore_barrier()

        pltpu.sync_copy(acc.at[pl.ds(zbase, SEG_PER_TILE)],
                        part_hbm.at[cid, pl.ds(zbase, SEG_PER_TILE)])

    return body(rows, ids, zeros)


def _tc_add_body(a_ref, b_ref, o_ref):
    o_ref[...] = a_ref[...] + b_ref[...]


def _tc_combine(parts):
    blk = 1000
    return pl.pallas_call(
        _tc_add_body,
        out_shape=jax.ShapeDtypeStruct((NSEG, D), jnp.float32),
        grid=(NSEG // blk,),
        in_specs=[pl.BlockSpec((blk, D), lambda i: (i, 0)),
                  pl.BlockSpec((blk, D), lambda i: (i, 0))],
        out_specs=pl.BlockSpec((blk, D), lambda i: (i, 0)),
    )(parts[0], parts[1])


def kernel(output, batch):
    ids = batch.astype(jnp.int32)
    zeros = jnp.zeros((NSEG_PAD, D), jnp.float32)
    parts = _sc_segment_partials(output, ids, zeros)
    return _tc_combine(parts).reshape(-1)


# speedup vs baseline: 6.8850x; 1.3827x over previous
import jax
import jax.numpy as jnp
from jax import lax
from jax.experimental import pallas as pl
from jax.experimental.pallas import tpu as pltpu
from jax.experimental.pallas import tpu_sc as plsc
import functools

N = 320000
D = 128
NSEG = 10000

NC = 2
NS = 16
NW = NC * NS
ROWS_PER_TILE = N // NW
CHUNK = 80
NCHUNK = ROWS_PER_TILE // CHUNK
NSEG_PAD = 10240
SEG_PER_TILE = NSEG_PAD // NS


def _sc_segment_partials(rows, ids, zeros):
    mesh = plsc.VectorSubcoreMesh(core_axis_name="c", subcore_axis_name="s")

    @functools.partial(
        pl.kernel,
        out_type=jax.ShapeDtypeStruct((NC, NSEG_PAD, D), jnp.float32),
        mesh=mesh,
        scratch_types=[
            pltpu.VMEM((CHUNK, D), jnp.float32),
            pltpu.VMEM((CHUNK, D), jnp.float32),
            pltpu.VMEM((CHUNK,), jnp.int32),
            pltpu.VMEM((CHUNK,), jnp.int32),
            pltpu.VMEM_SHARED((NSEG_PAD, D), jnp.float32),
            pltpu.SemaphoreType.DMA,
            pltpu.SemaphoreType.DMA,
            pltpu.SemaphoreType.DMA,
            pltpu.SemaphoreType.DMA,
        ],
    )
    def body(rows_hbm, ids_hbm, zeros_hbm, part_hbm,
             rows_v0, rows_v1, idx_v0, idx_v1, acc,
             rsem0, rsem1, isem0, isem1):
        cid = lax.axis_index("c")
        sid = lax.axis_index("s")
        base_row = (cid * NS + sid) * ROWS_PER_TILE
        zbase = sid * SEG_PER_TILE
        rows_v = (rows_v0, rows_v1)
        idx_v = (idx_v0, idx_v1)
        rsem = (rsem0, rsem1)
        isem = (isem0, isem1)

        def start(i, b):
            off = base_row + i * CHUNK
            pltpu.async_copy(ids_hbm.at[pl.ds(off, CHUNK)], idx_v[b], isem[b])
            pltpu.async_copy(rows_hbm.at[pl.ds(off, CHUNK)], rows_v[b], rsem[b])

        def wait(b):
            pltpu.make_async_copy(ids_hbm.at[pl.ds(0, CHUNK)], idx_v[b], isem[b]).wait()
            pltpu.make_async_copy(rows_hbm.at[pl.ds(0, CHUNK)], rows_v[b], rsem[b]).wait()

        def scatter(b):
            pltpu.sync_copy(rows_v[b], acc.at[idx_v[b]], add=True)

        pltpu.sync_copy(zeros_hbm.at[pl.ds(zbase, SEG_PER_TILE)],
                        acc.at[pl.ds(zbase, SEG_PER_TILE)])
        plsc.subcore_barrier()

        start(0, 0)

        def pair(p, carry):
            start(2 * p + 1, 1)
            wait(0)
            scatter(0)
            start(2 * p + 2, 0)
            wait(1)
            scatter(1)
            return carry

        lax.fori_loop(0, (NCHUNK - 1) // 2, pair, 0)
        wait(0)
        scatter(0)
        plsc.subcore_barrier()

        pltpu.sync_copy(acc.at[pl.ds(zbase, SEG_PER_TILE)],
                        part_hbm.at[cid, pl.ds(zbase, SEG_PER_TILE)])

    return body(rows, ids, zeros)


def _tc_add_body(a_ref, b_ref, o_ref):
    o_ref[...] = a_ref[...] + b_ref[...]


def _tc_combine(parts):
    blk = 1000
    return pl.pallas_call(
        _tc_add_body,
        out_shape=jax.ShapeDtypeStruct((NSEG, D), jnp.float32),
        grid=(NSEG // blk,),
        in_specs=[pl.BlockSpec((blk, D), lambda i: (i, 0)),
                  pl.BlockSpec((blk, D), lambda i: (i, 0))],
        out_specs=pl.BlockSpec((blk, D), lambda i: (i, 0)),
    )(parts[0], parts[1])


def kernel(output, batch):
    ids = batch.astype(jnp.int32)
    zeros = jnp.zeros((NSEG_PAD, D), jnp.float32)
    parts = _sc_segment_partials(output, ids, zeros)
    return _tc_combine(parts).reshape(-1)


# speedup vs baseline: 7.3422x; 1.0664x over previous
import jax
import jax.numpy as jnp
from jax import lax
from jax.experimental import pallas as pl
from jax.experimental.pallas import tpu as pltpu
from jax.experimental.pallas import tpu_sc as plsc
import functools

N = 320000
D = 128
NSEG = 10000

NC = 2
NS = 16
NW = NC * NS
ROWS_PER_TILE = N // NW
CHUNK = 192
NFULL = ROWS_PER_TILE // CHUNK
TAIL = ROWS_PER_TILE - NFULL * CHUNK
NSEG_PAD = 10112
SEG_PER_TILE = NSEG_PAD // NS


def _sc_segment_partials(rows, ids, zeros):
    mesh = plsc.VectorSubcoreMesh(core_axis_name="c", subcore_axis_name="s")

    @functools.partial(
        pl.kernel,
        out_type=jax.ShapeDtypeStruct((NC, NSEG_PAD, D), jnp.float32),
        mesh=mesh,
        scratch_types=[
            pltpu.VMEM((CHUNK, D), jnp.float32),
            pltpu.VMEM((CHUNK, D), jnp.float32),
            pltpu.VMEM((CHUNK,), jnp.int32),
            pltpu.VMEM((CHUNK,), jnp.int32),
            pltpu.VMEM_SHARED((NSEG_PAD, D), jnp.float32),
            pltpu.SemaphoreType.DMA,
            pltpu.SemaphoreType.DMA,
            pltpu.SemaphoreType.DMA,
            pltpu.SemaphoreType.DMA,
            pltpu.SemaphoreType.DMA,
        ],
    )
    def body(rows_hbm, ids_hbm, zeros_hbm, part_hbm,
             rows_v0, rows_v1, idx_v0, idx_v1, acc,
             rsem0, rsem1, isem0, isem1, zsem):
        cid = lax.axis_index("c")
        sid = lax.axis_index("s")
        base_row = (cid * NS + sid) * ROWS_PER_TILE
        zbase = sid * SEG_PER_TILE
        rows_v = (rows_v0, rows_v1)
        idx_v = (idx_v0, idx_v1)
        rsem = (rsem0, rsem1)
        isem = (isem0, isem1)

        def _sl(ref, size):
            return ref if size == CHUNK else ref.at[pl.ds(0, size)]

        def start(i, b, size=CHUNK):
            off = base_row + i * CHUNK
            pltpu.async_copy(ids_hbm.at[pl.ds(off, size)],
                             _sl(idx_v[b], size), isem[b])
            pltpu.async_copy(rows_hbm.at[pl.ds(off, size)],
                             _sl(rows_v[b], size), rsem[b])

        def wait(b, size=CHUNK):
            pltpu.make_async_copy(ids_hbm.at[pl.ds(0, size)],
                                  _sl(idx_v[b], size), isem[b]).wait()
            pltpu.make_async_copy(rows_hbm.at[pl.ds(0, size)],
                                  _sl(rows_v[b], size), rsem[b]).wait()

        def scatter(b, size=CHUNK):
            pltpu.sync_copy(_sl(rows_v[b], size),
                            acc.at[_sl(idx_v[b], size)], add=True)

        start(0, 0)
        zcopy = pltpu.async_copy(zeros_hbm.at[pl.ds(zbase, SEG_PER_TILE)],
                                 acc.at[pl.ds(zbase, SEG_PER_TILE)], zsem)
        zcopy.wait()
        plsc.subcore_barrier()

        def pair(p, carry):
            start(2 * p + 1, 1)
            wait(0)
            scatter(0)
            start(2 * p + 2, 0)
            wait(1)
            scatter(1)
            return carry

        lax.fori_loop(0, NFULL // 2 - 1, pair, 0)
        start(NFULL - 1, 1)
        wait(0)
        scatter(0)
        start(NFULL, 0, TAIL)
        wait(1)
        scatter(1)
        wait(0, TAIL)
        scatter(0, TAIL)
        plsc.subcore_barrier()

        pltpu.sync_copy(acc.at[pl.ds(zbase, SEG_PER_TILE)],
                        part_hbm.at[cid, pl.ds(zbase, SEG_PER_TILE)])

    return body(rows, ids, zeros)


def _tc_add_body(a_ref, b_ref, o_ref):
    o_ref[...] = a_ref[...] + b_ref[...]


def _tc_combine(parts):
    blk = 1000
    return pl.pallas_call(
        _tc_add_body,
        out_shape=jax.ShapeDtypeStruct((NSEG, D), jnp.float32),
        grid=(NSEG // blk,),
        in_specs=[pl.BlockSpec((blk, D), lambda i: (i, 0)),
                  pl.BlockSpec((blk, D), lambda i: (i, 0))],
        out_specs=pl.BlockSpec((blk, D), lambda i: (i, 0)),
    )(parts[0], parts[1])


def kernel(output, batch):
    ids = batch.astype(jnp.int32)
    zeros = jnp.zeros((NSEG_PAD, D), jnp.float32)
    parts = _sc_segment_partials(output, ids, zeros)
    return _tc_combine(parts).reshape(-1)
